# baseline (speedup 1.0000x reference)
import jax
import jax.numpy as jnp
from jax import lax
from jax.experimental import pallas as pl
from jax.experimental.pallas import tpu as pltpu
from jax.experimental.pallas import tpu_sc as plsc

_NUM_ROWS = 64
_ROW_STRIDE = 700
_D = 256
_NC = 2
_SC_ROWS = 2
_TC_ROWS = _NUM_ROWS - _SC_ROWS
_ROWS_PER_CORE = _SC_ROWS // _NC


def _sc_body(table_hbm, out_hbm, buf_spmem, sem):
    cid = lax.axis_index("c")
    base = cid * _ROWS_PER_CORE
    copies = []
    for j in range(_ROWS_PER_CORE):
        copies.append(
            pltpu.make_async_copy(
                table_hbm.at[pl.ds((base + j) * _ROW_STRIDE, 1)],
                buf_spmem.at[pl.ds(j, 1)],
                sem,
            )
        )
    for c in copies:
        c.start()
    for c in copies:
        c.wait()
    pltpu.sync_copy(buf_spmem, out_hbm.at[pl.ds(base, _ROWS_PER_CORE)])


def _tc_body(table_hbm, out_hbm, sem):
    copies = []
    for j in range(_TC_ROWS):
        copies.append(
            pltpu.make_async_copy(
                table_hbm.at[pl.ds((_SC_ROWS + j) * _ROW_STRIDE, 1)],
                out_hbm.at[pl.ds(j, 1)],
                sem,
            )
        )
    for c in copies:
        c.start()
    for c in copies:
        c.wait()


def kernel(inputs):
    parts = []
    if _SC_ROWS:
        parts.append(pl.kernel(
            _sc_body,
            out_type=jax.ShapeDtypeStruct((_SC_ROWS, _D), jnp.float32),
            mesh=plsc.ScalarSubcoreMesh(axis_name="c", num_cores=_NC),
            scratch_types=[
                pltpu.VMEM_SHARED((_ROWS_PER_CORE, _D), jnp.float32),
                pltpu.SemaphoreType.DMA,
            ],
        )(inputs))
    if _TC_ROWS:
        parts.append(pl.pallas_call(
            _tc_body,
            out_shape=jax.ShapeDtypeStruct((_TC_ROWS, _D), jnp.float32),
            in_specs=[pl.BlockSpec(memory_space=pltpu.MemorySpace.HBM)],
            out_specs=pl.BlockSpec(memory_space=pltpu.MemorySpace.HBM),
            scratch_shapes=[pltpu.SemaphoreType.DMA],
        )(inputs))
    gathered = parts[0] if len(parts) == 1 else jnp.concatenate(parts, axis=0)
    return jnp.reshape(gathered, (1, _NUM_ROWS * _D))


# speedup vs baseline: 6.2521x; 6.2521x over previous
import jax
import jax.numpy as jnp
from jax.experimental import pallas as pl
from jax.experimental.pallas import tpu as pltpu

_NUM_ROWS = 64
_ROW_STRIDE = 700
_D = 256


def _tc_body(table_hbm, out_vmem, sem):
    copies = []
    for j in range(_NUM_ROWS):
        copies.append(
            pltpu.make_async_copy(
                table_hbm.at[pl.ds(j * _ROW_STRIDE, 1)],
                out_vmem.at[pl.ds(j, 1)],
                sem,
            )
        )
    for c in copies:
        c.start()
    for c in copies:
        c.wait()


def kernel(inputs):
    gathered = pl.pallas_call(
        _tc_body,
        out_shape=jax.ShapeDtypeStruct((_NUM_ROWS, _D), jnp.float32),
        in_specs=[pl.BlockSpec(memory_space=pltpu.MemorySpace.HBM)],
        out_specs=pl.BlockSpec(memory_space=pltpu.MemorySpace.VMEM),
        scratch_shapes=[pltpu.SemaphoreType.DMA],
    )(inputs)
    return jnp.reshape(gathered, (1, _NUM_ROWS * _D))
